# Initial kernel scaffold; baseline (speedup 1.0000x reference)
import jax
import jax.numpy as jnp
from jax.experimental import pallas as pl


def kernel(in_adjacent_list, out_adjacent_list, W1, b1, W2, b2, W3, b3, W4, b4, LW1, LB1, LW2, LB2, LW3, LB3):
    raise NotImplementedError("write your pallas kernel here")


# speedup vs baseline: 1.2024x; 1.2024x over previous
import jax
import jax.numpy as jnp
from jax import lax
from jax.experimental import pallas as pl
from jax.experimental.pallas import tpu as pltpu

F32 = jnp.float32
BF16 = jnp.bfloat16


def _pick_bm(n):
    for bm in (400, 200, 100, 50, 40, 25, 16, 10, 8):
        if n % bm == 0:
            return bm
    return n


def _mmb(x, wb):
    return jnp.dot(x.astype(BF16), wb, preferred_element_type=F32)


def _l2normalize(x):
    n = jnp.sqrt(jnp.sum(x * x, axis=1, keepdims=True))
    return x / jnp.maximum(n, 1e-12)


def _score(x, h):
    (lw1, lb1, lw2, lb2, lw3, lb3) = h
    s = jax.nn.relu(_mmb(x, lw1[...]) + lb1[...])
    s = jax.nn.relu(_mmb(s, lw2[...]) + lb2[...])
    return _mmb(s, lw3[...]) + lb3[...]


def _layer1_body(adj_ref, w1_ref, b1_ref, wn_ref, *rest):
    (lw1, lb1, lw2, lb2, lw3, lb3, ab_ref, y_ref, s_ref) = rest
    a = adj_ref[...].astype(BF16)
    ab_ref[...] = a
    z = jnp.dot(a, w1_ref[...], preferred_element_type=F32)
    x = _l2normalize(jax.nn.relu(z + b1_ref[...]))
    s_ref[...] = _score(x, (lw1, lb1, lw2, lb2, lw3, lb3))
    y_ref[...] = _mmb(x, wn_ref[...]).astype(BF16)


def _mid_body(ab_ref, y_ref, sp_ref, b_ref, wn_ref, *rest):
    (lw1, lb1, lw2, lb2, lw3, lb3, yn_ref, s_ref) = rest
    z = jnp.dot(ab_ref[...], y_ref[...], preferred_element_type=F32)
    x = _l2normalize(jax.nn.relu(z + b_ref[...]))
    s_ref[...] = sp_ref[...] + _score(x, (lw1, lb1, lw2, lb2, lw3, lb3))
    yn_ref[...] = _mmb(x, wn_ref[...]).astype(BF16)


def _last_body_first(ab_ref, y_ref, sp_ref, b_ref, *rest):
    (lw1, lb1, lw2, lb2, lw3, lb3, s_ref) = rest
    z = jnp.dot(ab_ref[...], y_ref[...], preferred_element_type=F32)
    x = jax.nn.relu(z + b_ref[...])
    s_ref[...] = sp_ref[...] + _score(x, (lw1, lb1, lw2, lb2, lw3, lb3))


def _last_body_mult(ab_ref, y_ref, sp_ref, sin_ref, b_ref, *rest):
    (lw1, lb1, lw2, lb2, lw3, lb3, out_ref) = rest
    z = jnp.dot(ab_ref[...], y_ref[...], preferred_element_type=F32)
    x = jax.nn.relu(z + b_ref[...])
    s = sp_ref[...] + _score(x, (lw1, lb1, lw2, lb2, lw3, lb3))
    out_ref[...] = s * sin_ref[...]


def _row_spec(bm, cols):
    return pl.BlockSpec((bm, cols), lambda i: (i, 0))


def _full_spec(shape):
    return pl.BlockSpec(shape, lambda i: tuple(0 for _ in shape))


def _compiler_params():
    return pltpu.CompilerParams(dimension_semantics=("arbitrary",),
                                vmem_limit_bytes=112 * 1024 * 1024)


def _layer1(adj, w1b, b1, wnb, heads, bm):
    n = adj.shape[0]
    h = w1b.shape[1]
    small = [w1b, b1, wnb] + list(heads)
    return pl.pallas_call(
        _layer1_body,
        grid=(n // bm,),
        in_specs=[_row_spec(bm, n)] + [_full_spec(a.shape) for a in small],
        out_specs=[_row_spec(bm, n), _row_spec(bm, h), _row_spec(bm, 1)],
        out_shape=[
            jax.ShapeDtypeStruct((n, n), BF16),
            jax.ShapeDtypeStruct((n, h), BF16),
            jax.ShapeDtypeStruct((n, 1), F32),
        ],
        compiler_params=_compiler_params(),
    )(adj, *small)


def _mid(ab, y, sp, b, wnb, heads, bm):
    n = ab.shape[0]
    h = y.shape[1]
    small = [b, wnb] + list(heads)
    return pl.pallas_call(
        _mid_body,
        grid=(n // bm,),
        in_specs=[_row_spec(bm, n), _full_spec(y.shape), _row_spec(bm, 1)]
                 + [_full_spec(a.shape) for a in small],
        out_specs=[_row_spec(bm, h), _row_spec(bm, 1)],
        out_shape=[
            jax.ShapeDtypeStruct((n, h), BF16),
            jax.ShapeDtypeStruct((n, 1), F32),
        ],
        compiler_params=_compiler_params(),
    )(ab, y, sp, *small)


def _last_first(ab, y, sp, b, heads, bm):
    n = ab.shape[0]
    small = [b] + list(heads)
    return pl.pallas_call(
        _last_body_first,
        grid=(n // bm,),
        in_specs=[_row_spec(bm, n), _full_spec(y.shape), _row_spec(bm, 1)]
                 + [_full_spec(a.shape) for a in small],
        out_specs=[_row_spec(bm, 1)],
        out_shape=[jax.ShapeDtypeStruct((n, 1), F32)],
        compiler_params=_compiler_params(),
    )(ab, y, sp, *small)[0]


def _last_mult(ab, y, sp, sin, b, heads, bm):
    n = ab.shape[0]
    small = [b] + list(heads)
    return pl.pallas_call(
        _last_body_mult,
        grid=(n // bm,),
        in_specs=[_row_spec(bm, n), _full_spec(y.shape), _row_spec(bm, 1),
                  _row_spec(bm, 1)] + [_full_spec(a.shape) for a in small],
        out_specs=[_row_spec(bm, 1)],
        out_shape=[jax.ShapeDtypeStruct((n, 1), F32)],
        compiler_params=_compiler_params(),
    )(ab, y, sp, sin, *small)[0]


def kernel(in_adjacent_list, out_adjacent_list, W1, b1, W2, b2, W3, b3,
           W4, b4, LW1, LB1, LW2, LB2, LW3, LB3):
    n = in_adjacent_list.shape[0]
    bm = _pick_bm(n)

    w1b = W1.astype(BF16)
    w2b = W2.astype(BF16)
    w3b = W3.astype(BF16)
    w4b = W4.astype(BF16)
    b1r, b2r, b3r, b4r = (b.reshape(1, -1) for b in (b1, b2, b3, b4))
    heads = (LW1.astype(BF16), LB1.reshape(1, -1), LW2.astype(BF16),
             LB2.reshape(1, -1), LW3.astype(BF16), LB3.reshape(1, -1))

    def branch_front(adj):
        ab, y2, s1 = _layer1(adj, w1b, b1r, w2b, heads, bm)
        y3, s2 = _mid(ab, y2, s1, b2r, w3b, heads, bm)
        y4, s3 = _mid(ab, y3, s2, b3r, w4b, heads, bm)
        return ab, y4, s3

    ab_in, y_in, sp_in = branch_front(in_adjacent_list)
    ab_out, y_out, sp_out = branch_front(out_adjacent_list)
    s_in = _last_first(ab_in, y_in, sp_in, b4r, heads, bm)
    return _last_mult(ab_out, y_out, sp_out, s_in, b4r, heads, bm)


# speedup vs baseline: 1.2286x; 1.0218x over previous
import jax
import jax.numpy as jnp
from jax import lax
from jax.experimental import pallas as pl
from jax.experimental.pallas import tpu as pltpu

F32 = jnp.float32
BF16 = jnp.bfloat16


def _pick_bm(n, target):
    for bm in (target, 1000, 400, 200, 100, 50, 40, 25, 16, 10, 8):
        if bm <= target and n % bm == 0:
            return bm
    return n


def _mmb(x, wb):
    return jnp.dot(x.astype(BF16), wb, preferred_element_type=F32)


def _l2normalize(x):
    n = jnp.sqrt(jnp.sum(x * x, axis=1, keepdims=True))
    return x / jnp.maximum(n, 1e-12)


def _score(x, h):
    (lw1, lb1, lw2, lb2, lw3, lb3) = h
    s = jax.nn.relu(_mmb(x, lw1[...]) + lb1[...])
    s = jax.nn.relu(_mmb(s, lw2[...]) + lb2[...])
    return _mmb(s, lw3[...]) + lb3[...]


def _layer1_body(adj_ref, w1_ref, b1_ref, wn_ref, *rest):
    (lw1, lb1, lw2, lb2, lw3, lb3, ab_ref, y_ref, s_ref) = rest
    a = adj_ref[...].astype(BF16)
    ab_ref[...] = a
    z = jnp.dot(a, w1_ref[...], preferred_element_type=F32)
    x = _l2normalize(jax.nn.relu(z + b1_ref[...]))
    s_ref[...] = _score(x, (lw1, lb1, lw2, lb2, lw3, lb3))
    y_ref[...] = _mmb(x, wn_ref[...]).astype(BF16)


def _mid_body(ab_ref, y_ref, sp_ref, b_ref, wn_ref, *rest):
    (lw1, lb1, lw2, lb2, lw3, lb3, yn_ref, s_ref) = rest
    z = jnp.dot(ab_ref[...], y_ref[...], preferred_element_type=F32)
    x = _l2normalize(jax.nn.relu(z + b_ref[...]))
    s_ref[...] = sp_ref[...] + _score(x, (lw1, lb1, lw2, lb2, lw3, lb3))
    yn_ref[...] = _mmb(x, wn_ref[...]).astype(BF16)


def _last_body_first(ab_ref, y_ref, sp_ref, b_ref, *rest):
    (lw1, lb1, lw2, lb2, lw3, lb3, s_ref) = rest
    z = jnp.dot(ab_ref[...], y_ref[...], preferred_element_type=F32)
    x = jax.nn.relu(z + b_ref[...])
    s_ref[...] = sp_ref[...] + _score(x, (lw1, lb1, lw2, lb2, lw3, lb3))


def _last_body_mult(ab_ref, y_ref, sp_ref, sin_ref, b_ref, *rest):
    (lw1, lb1, lw2, lb2, lw3, lb3, out_ref) = rest
    z = jnp.dot(ab_ref[...], y_ref[...], preferred_element_type=F32)
    x = jax.nn.relu(z + b_ref[...])
    s = sp_ref[...] + _score(x, (lw1, lb1, lw2, lb2, lw3, lb3))
    out_ref[...] = s * sin_ref[...]


def _row_spec(bm, cols):
    return pl.BlockSpec((bm, cols), lambda i: (i, 0))


def _full_spec(shape):
    return pl.BlockSpec(shape, lambda i: tuple(0 for _ in shape))


def _compiler_params():
    return pltpu.CompilerParams(dimension_semantics=("parallel",),
                                vmem_limit_bytes=112 * 1024 * 1024)


def _layer1(adj, w1b, b1, wnb, heads, bm):
    n = adj.shape[0]
    h = w1b.shape[1]
    small = [w1b, b1, wnb] + list(heads)
    return pl.pallas_call(
        _layer1_body,
        grid=(n // bm,),
        in_specs=[_row_spec(bm, n)] + [_full_spec(a.shape) for a in small],
        out_specs=[_row_spec(bm, n), _row_spec(bm, h), _row_spec(bm, 1)],
        out_shape=[
            jax.ShapeDtypeStruct((n, n), BF16),
            jax.ShapeDtypeStruct((n, h), BF16),
            jax.ShapeDtypeStruct((n, 1), F32),
        ],
        compiler_params=_compiler_params(),
    )(adj, *small)


def _mid(ab, y, sp, b, wnb, heads, bm):
    n = ab.shape[0]
    h = y.shape[1]
    small = [b, wnb] + list(heads)
    return pl.pallas_call(
        _mid_body,
        grid=(n // bm,),
        in_specs=[_row_spec(bm, n), _full_spec(y.shape), _row_spec(bm, 1)]
                 + [_full_spec(a.shape) for a in small],
        out_specs=[_row_spec(bm, h), _row_spec(bm, 1)],
        out_shape=[
            jax.ShapeDtypeStruct((n, h), BF16),
            jax.ShapeDtypeStruct((n, 1), F32),
        ],
        compiler_params=_compiler_params(),
    )(ab, y, sp, *small)


def _last_first(ab, y, sp, b, heads, bm):
    n = ab.shape[0]
    small = [b] + list(heads)
    return pl.pallas_call(
        _last_body_first,
        grid=(n // bm,),
        in_specs=[_row_spec(bm, n), _full_spec(y.shape), _row_spec(bm, 1)]
                 + [_full_spec(a.shape) for a in small],
        out_specs=[_row_spec(bm, 1)],
        out_shape=[jax.ShapeDtypeStruct((n, 1), F32)],
        compiler_params=_compiler_params(),
    )(ab, y, sp, *small)[0]


def _last_mult(ab, y, sp, sin, b, heads, bm):
    n = ab.shape[0]
    small = [b] + list(heads)
    return pl.pallas_call(
        _last_body_mult,
        grid=(n // bm,),
        in_specs=[_row_spec(bm, n), _full_spec(y.shape), _row_spec(bm, 1),
                  _row_spec(bm, 1)] + [_full_spec(a.shape) for a in small],
        out_specs=[_row_spec(bm, 1)],
        out_shape=[jax.ShapeDtypeStruct((n, 1), F32)],
        compiler_params=_compiler_params(),
    )(ab, y, sp, sin, *small)[0]


def kernel(in_adjacent_list, out_adjacent_list, W1, b1, W2, b2, W3, b3,
           W4, b4, LW1, LB1, LW2, LB2, LW3, LB3):
    n = in_adjacent_list.shape[0]
    bm = _pick_bm(n, 400)
    bmm = _pick_bm(n, 1000)

    w1b = W1.astype(BF16)
    w2b = W2.astype(BF16)
    w3b = W3.astype(BF16)
    w4b = W4.astype(BF16)
    b1r, b2r, b3r, b4r = (b.reshape(1, -1) for b in (b1, b2, b3, b4))
    heads = (LW1.astype(BF16), LB1.reshape(1, -1), LW2.astype(BF16),
             LB2.reshape(1, -1), LW3.astype(BF16), LB3.reshape(1, -1))

    def branch_front(adj):
        ab, y2, s1 = _layer1(adj, w1b, b1r, w2b, heads, bm)
        y3, s2 = _mid(ab, y2, s1, b2r, w3b, heads, bmm)
        y4, s3 = _mid(ab, y3, s2, b3r, w4b, heads, bmm)
        return ab, y4, s3

    ab_in, y_in, sp_in = branch_front(in_adjacent_list)
    ab_out, y_out, sp_out = branch_front(out_adjacent_list)
    s_in = _last_first(ab_in, y_in, sp_in, b4r, heads, bmm)
    return _last_mult(ab_out, y_out, sp_out, s_in, b4r, heads, bmm)
